# baseline (speedup 1.0000x reference)
import functools

import jax
import jax.numpy as jnp
import numpy as np
from jax import lax
from jax.experimental import pallas as pl
from jax.experimental.pallas import tpu as pltpu
from jax.experimental.pallas import tpu_sc as plsc

N = 10000
E = 320000
D = 128
ETYPES = 4
STEPS = 8
NUM_GRAPHS = 10
NODES_PER_GRAPH = 1000
NCLS = 46

BN = 2000

DH = D // 2
CHUNK = 128
NCH = 157
NBUF = 6
PREF = 5
EPAD = 16 * NCH * CHUNK
SLAB = 632
ACC_ROWS = N + 8
ZTAIL = ACC_ROWS - 15 * SLAB
TAIL = N - 15 * SLAB

_S2F = np.empty(D, np.int32)
for _h in range(2):
    for _g in range(2):
        for _k in range(16):
            _S2F[64 * _h + 32 * _g + _k] = 64 * _h + 32 * _g + 2 * _k
            _S2F[64 * _h + 32 * _g + 16 + _k] = 64 * _h + 32 * _g + 2 * _k + 1


def _first_trans_body(x_ref, we_ref, be_ref, trans_ref):
    h = x_ref[...]
    for t in range(ETYPES):
        tr = jnp.dot(h, we_ref[t], preferred_element_type=jnp.float32)
        trans_ref[t] = (tr + be_ref[t][None, :]).astype(jnp.bfloat16)


def _first_trans(x, W_edge, b_edge):
    grid = (N // BN,)
    return pl.pallas_call(
        _first_trans_body,
        grid=grid,
        in_specs=[
            pl.BlockSpec((BN, D), lambda i: (i, 0)),
            pl.BlockSpec((ETYPES, D, D), lambda i: (0, 0, 0)),
            pl.BlockSpec((ETYPES, D), lambda i: (0, 0)),
        ],
        out_specs=pl.BlockSpec((ETYPES, BN, D), lambda i: (0, i, 0)),
        out_shape=jax.ShapeDtypeStruct((ETYPES, N, D), jnp.bfloat16),
    )(x, W_edge, b_edge)


def _gru_body(with_trans, part_ref, h_ref, wih_ref, whh_ref, bih_ref,
              bhh_ref, we_ref, be_ref, h_new_ref, trans_ref=None):
    a = jnp.concatenate([part_ref[0], part_ref[1]], axis=1)
    h = h_ref[...]
    gi = jnp.dot(a, wih_ref[...], preferred_element_type=jnp.float32) + bih_ref[...]
    gh = jnp.dot(h, whh_ref[...], preferred_element_type=jnp.float32) + bhh_ref[...]
    r = jax.nn.sigmoid(gi[:, :D] + gh[:, :D])
    z = jax.nn.sigmoid(gi[:, D:2 * D] + gh[:, D:2 * D])
    n = jnp.tanh(gi[:, 2 * D:] + r * gh[:, 2 * D:])
    h_new = (1.0 - z) * n + z * h
    h_new_ref[...] = h_new
    if with_trans:
        for t in range(ETYPES):
            tr = jnp.dot(h_new, we_ref[t], preferred_element_type=jnp.float32)
            trans_ref[t] = (tr + be_ref[t][None, :]).astype(jnp.bfloat16)


def _gru_step(part, h, W_ih_T, W_hh_T, b_ih2, b_hh2, W_edge, b_edge,
              with_trans):
    grid = (N // BN,)
    out_shape = [jax.ShapeDtypeStruct((N, D), jnp.float32)]
    out_specs = [pl.BlockSpec((BN, D), lambda i: (i, 0))]
    if with_trans:
        out_shape.append(jax.ShapeDtypeStruct((ETYPES, N, D), jnp.bfloat16))
        out_specs.append(pl.BlockSpec((ETYPES, BN, D), lambda i: (0, i, 0)))
    res = pl.pallas_call(
        functools.partial(_gru_body, with_trans),
        grid=grid,
        in_specs=[
            pl.BlockSpec((2, BN, DH), lambda i: (0, i, 0)),
            pl.BlockSpec((BN, D), lambda i: (i, 0)),
            pl.BlockSpec((D, 3 * D), lambda i: (0, 0)),
            pl.BlockSpec((D, 3 * D), lambda i: (0, 0)),
            pl.BlockSpec((1, 3 * D), lambda i: (0, 0)),
            pl.BlockSpec((1, 3 * D), lambda i: (0, 0)),
            pl.BlockSpec((ETYPES, D, D), lambda i: (0, 0, 0)),
            pl.BlockSpec((ETYPES, D), lambda i: (0, 0)),
        ],
        out_specs=out_specs,
        out_shape=out_shape,
    )(part, h, W_ih_T, W_hh_T, b_ih2, b_hh2, W_edge, b_edge)
    if with_trans:
        return res[0], res[1]
    return res[0], None


def _final_body(h_ref, wcls_ref, bcls_ref, hi_ref, cls_ref):
    y = jnp.tanh(h_ref[...])
    hi_ref[0] = y
    s = jnp.sum(y, axis=0, keepdims=True)
    cls_ref[0] = jnp.dot(s, wcls_ref[...].T,
                         preferred_element_type=jnp.float32) + bcls_ref[...]


def _final(h, W_cls, b_cls2):
    grid = (NUM_GRAPHS,)
    return pl.pallas_call(
        _final_body,
        grid=grid,
        in_specs=[
            pl.BlockSpec((NODES_PER_GRAPH, D), lambda i: (i, 0)),
            pl.BlockSpec((NCLS, D), lambda i: (0, 0)),
            pl.BlockSpec((1, NCLS), lambda i: (0, 0)),
        ],
        out_specs=[
            pl.BlockSpec((1, NODES_PER_GRAPH, D), lambda i: (i, 0, 0)),
            pl.BlockSpec((1, 1, NCLS), lambda i: (i, 0, 0)),
        ],
        out_shape=[
            jax.ShapeDtypeStruct((NUM_GRAPHS, NODES_PER_GRAPH, D), jnp.float32),
            jax.ShapeDtypeStruct((NUM_GRAPHS, 1, NCLS), jnp.float32),
        ],
    )(h, W_cls, b_cls2)


def _sc_edge_body(trans_hbm, packed_hbm, out_hbm,
                  packed_v, idxs_v, dsts_v, rows_bf, rowsf_v, acc_sh,
                  gsem, ssem):
    c = lax.axis_index("c")
    s = lax.axis_index("s")

    def _z(i, _):
        for j in range(DH // 16):
            rowsf_v[0, i, pl.ds(j * 16, 16)] = jnp.zeros((16,), jnp.float32)
        return 0
    lax.fori_loop(0, CHUNK, _z, 0)
    base = pl.multiple_of(s * SLAB, 8)

    def _zk(k, _):
        pltpu.sync_copy(rowsf_v.at[0],
                        acc_sh.at[pl.ds(base + k * CHUNK, CHUNK)])
        return 0
    lax.fori_loop(0, SLAB // CHUNK, _zk, 0)

    @pl.when(s < 15)
    def _zremf():
        zr = SLAB % CHUNK
        pltpu.sync_copy(rowsf_v.at[0, pl.ds(0, zr)],
                        acc_sh.at[pl.ds(base + SLAB - zr, zr)])

    @pl.when(s == 15)
    def _zremt():
        zr = ZTAIL % CHUNK
        pltpu.sync_copy(rowsf_v.at[0, pl.ds(0, zr)],
                        acc_sh.at[pl.ds(15 * SLAB + ZTAIL - zr, zr)])

    pltpu.sync_copy(packed_hbm.at[s], packed_v)

    plsc.subcore_barrier()

    def _unpack(jj, slot):
        for i in range(CHUNK // 16):
            v = packed_v[jj, pl.ds(i * 16, 16)]
            key16 = jnp.bitwise_and(v, 0xFFFF)
            idxs_v[slot, pl.ds(i * 16, 16)] = key16 * 2 + c
            dsts_v[slot, pl.ds(i * 16, 16)] = lax.shift_right_logical(v, 16)

    def _fire_gather(jj, slot):
        pltpu.async_copy(trans_hbm.at[idxs_v.at[slot]], rows_bf.at[slot],
                         gsem)

    def _convert(slot, slot3):
        def _cr(row, _):
            for g in range(DH // 32):
                v = rows_bf[slot, row, pl.ds(g * 32, 32)]
                p = plsc.bitcast(v, jnp.int32)
                e = plsc.bitcast(lax.shift_left(p, 16), jnp.float32)
                o = plsc.bitcast(jnp.bitwise_and(p, jnp.int32(-65536)),
                                 jnp.float32)
                rowsf_v[slot3, row, pl.ds(g * 32, 16)] = e
                rowsf_v[slot3, row, pl.ds(g * 32 + 16, 16)] = o
            return 0
        lax.fori_loop(0, CHUNK, _cr, 0)

    def _prime(p, _):
        _unpack(p, p)
        _fire_gather(p, p)
        return 0
    lax.fori_loop(0, PREF, _prime, 0)

    def _chunk(j, _):
        r = j % NBUF
        r3 = j % 3
        rn = (j + PREF) % NBUF
        pltpu.make_async_copy(trans_hbm.at[idxs_v.at[r]], rows_bf.at[r],
                              gsem).wait()
        _convert(r, r3)
        pltpu.async_copy(rowsf_v.at[r3], acc_sh.at[dsts_v.at[r]], ssem,
                         add=True)

        @pl.when(j >= 1)
        def _reclaim():
            pltpu.make_async_copy(rowsf_v.at[(j - 1) % 3],
                                  acc_sh.at[dsts_v.at[(j - 1) % NBUF]],
                                  ssem).wait()

        @pl.when(j + PREF < NCH)
        def _prefetch():
            _unpack(j + PREF, rn)
            _fire_gather(j + PREF, rn)
        return 0
    lax.fori_loop(0, NCH, _chunk, 0)

    pltpu.make_async_copy(rowsf_v.at[(NCH - 1) % 3],
                          acc_sh.at[dsts_v.at[(NCH - 1) % NBUF]],
                          ssem).wait()

    plsc.subcore_barrier()

    @pl.when(s < 15)
    def _full():
        pltpu.sync_copy(acc_sh.at[pl.ds(base, SLAB)],
                        out_hbm.at[c, pl.ds(base, SLAB)])

    @pl.when(s == 15)
    def _tail():
        pltpu.sync_copy(acc_sh.at[pl.ds(15 * SLAB, TAIL)],
                        out_hbm.at[c, pl.ds(15 * SLAB, TAIL)])


_sc_edge = functools.partial(
    pl.kernel,
    out_type=jax.ShapeDtypeStruct((2, N, DH), jnp.float32),
    mesh=plsc.VectorSubcoreMesh(core_axis_name="c", subcore_axis_name="s"),
    compiler_params=pltpu.CompilerParams(use_tc_tiling_on_sc=False,
                                         needs_layout_passes=False),
    scratch_types=[
        pltpu.VMEM((NCH, CHUNK), jnp.int32),
        pltpu.VMEM((NBUF, CHUNK), jnp.int32),
        pltpu.VMEM((NBUF, CHUNK), jnp.int32),
        pltpu.VMEM((NBUF, CHUNK, DH), jnp.bfloat16),
        pltpu.VMEM((3, CHUNK, DH), jnp.float32),
        pltpu.VMEM_SHARED((ACC_ROWS, DH), jnp.float32),
        pltpu.SemaphoreType.DMA,
        pltpu.SemaphoreType.DMA,
    ],
)(_sc_edge_body)


def kernel(x, edge_index, edge_types, W_edge, b_edge, W_ih, W_hh, b_ih, b_hh,
           W_cls, b_cls):
    src = edge_index[0]
    dst = edge_index[1]
    key = edge_types * N + src
    packed = key + (dst << 16)
    packed_r = jnp.concatenate(
        [packed, jnp.full((EPAD - E,), N << 16, jnp.int32)]
    ).reshape(16, NCH, CHUNK)

    W_ih_T = W_ih.T[_S2F, :]
    W_hh_T = W_hh.T
    b_ih2 = b_ih[None, :]
    b_hh2 = b_hh[None, :]
    b_cls2 = b_cls[None, :]

    h = x
    trans = _first_trans(x, W_edge, b_edge)
    for step in range(STEPS):
        table = trans.reshape(2 * ETYPES * N, DH)
        part = _sc_edge(table, packed_r)
        h, trans = _gru_step(part, h, W_ih_T, W_hh_T, b_ih2,
                             b_hh2, W_edge, b_edge,
                             with_trans=(step < STEPS - 1))
    h_i, ggnn_sum = _final(h, W_cls, b_cls2)
    return (ggnn_sum.reshape(NUM_GRAPHS, NCLS), h_i)


# speedup vs baseline: 1.7809x; 1.7809x over previous
import functools

import jax
import jax.numpy as jnp
from jax import lax
from jax.experimental import pallas as pl
from jax.experimental.pallas import tpu as pltpu
from jax.experimental.pallas import tpu_sc as plsc

N = 10000
E = 320000
D = 128
ETYPES = 4
STEPS = 8
NUM_GRAPHS = 10
NODES_PER_GRAPH = 1000
NCLS = 46

BN = 2000

DH = D // 2
CHUNK = 128
NCH = 157
NBUF = 7
PREF = 6
EPAD = 16 * NCH * CHUNK
SLAB = 632
ACC_ROWS = N + 8
ZTAIL = ACC_ROWS - 15 * SLAB
TAIL = N - 15 * SLAB


def _first_trans_body(x_ref, we_ref, be_ref, trans_ref):
    h = x_ref[...]
    for t in range(ETYPES):
        tr = jnp.dot(h, we_ref[t], preferred_element_type=jnp.float32)
        trans_ref[t] = tr + be_ref[t][None, :]


def _first_trans(x, W_edge, b_edge):
    grid = (N // BN,)
    return pl.pallas_call(
        _first_trans_body,
        grid=grid,
        in_specs=[
            pl.BlockSpec((BN, D), lambda i: (i, 0)),
            pl.BlockSpec((ETYPES, D, D), lambda i: (0, 0, 0)),
            pl.BlockSpec((ETYPES, D), lambda i: (0, 0)),
        ],
        out_specs=pl.BlockSpec((ETYPES, BN, D), lambda i: (0, i, 0)),
        out_shape=jax.ShapeDtypeStruct((ETYPES, N, D), jnp.float32),
    )(x, W_edge, b_edge)


def _gru_body(with_trans, part_ref, h_ref, wih_ref, whh_ref, bih_ref,
              bhh_ref, we_ref, be_ref, h_new_ref, trans_ref=None):
    a = jnp.concatenate([part_ref[0], part_ref[1]], axis=1)
    h = h_ref[...]
    gi = jnp.dot(a, wih_ref[...], preferred_element_type=jnp.float32) + bih_ref[...]
    gh = jnp.dot(h, whh_ref[...], preferred_element_type=jnp.float32) + bhh_ref[...]
    r = jax.nn.sigmoid(gi[:, :D] + gh[:, :D])
    z = jax.nn.sigmoid(gi[:, D:2 * D] + gh[:, D:2 * D])
    n = jnp.tanh(gi[:, 2 * D:] + r * gh[:, 2 * D:])
    h_new = (1.0 - z) * n + z * h
    h_new_ref[...] = h_new
    if with_trans:
        for t in range(ETYPES):
            tr = jnp.dot(h_new, we_ref[t], preferred_element_type=jnp.float32)
            trans_ref[t] = tr + be_ref[t][None, :]


def _gru_step(part, h, W_ih_T, W_hh_T, b_ih2, b_hh2, W_edge, b_edge,
              with_trans):
    grid = (N // BN,)
    out_shape = [jax.ShapeDtypeStruct((N, D), jnp.float32)]
    out_specs = [pl.BlockSpec((BN, D), lambda i: (i, 0))]
    if with_trans:
        out_shape.append(jax.ShapeDtypeStruct((ETYPES, N, D), jnp.float32))
        out_specs.append(pl.BlockSpec((ETYPES, BN, D), lambda i: (0, i, 0)))
    res = pl.pallas_call(
        functools.partial(_gru_body, with_trans),
        grid=grid,
        in_specs=[
            pl.BlockSpec((2, BN, DH), lambda i: (0, i, 0)),
            pl.BlockSpec((BN, D), lambda i: (i, 0)),
            pl.BlockSpec((D, 3 * D), lambda i: (0, 0)),
            pl.BlockSpec((D, 3 * D), lambda i: (0, 0)),
            pl.BlockSpec((1, 3 * D), lambda i: (0, 0)),
            pl.BlockSpec((1, 3 * D), lambda i: (0, 0)),
            pl.BlockSpec((ETYPES, D, D), lambda i: (0, 0, 0)),
            pl.BlockSpec((ETYPES, D), lambda i: (0, 0)),
        ],
        out_specs=out_specs,
        out_shape=out_shape,
    )(part, h, W_ih_T, W_hh_T, b_ih2, b_hh2, W_edge, b_edge)
    if with_trans:
        return res[0], res[1]
    return res[0], None


def _final_body(h_ref, wcls_ref, bcls_ref, hi_ref, cls_ref):
    y = jnp.tanh(h_ref[...])
    hi_ref[0] = y
    s = jnp.sum(y, axis=0, keepdims=True)
    cls_ref[0] = jnp.dot(s, wcls_ref[...].T,
                         preferred_element_type=jnp.float32) + bcls_ref[...]


def _final(h, W_cls, b_cls2):
    grid = (NUM_GRAPHS,)
    return pl.pallas_call(
        _final_body,
        grid=grid,
        in_specs=[
            pl.BlockSpec((NODES_PER_GRAPH, D), lambda i: (i, 0)),
            pl.BlockSpec((NCLS, D), lambda i: (0, 0)),
            pl.BlockSpec((1, NCLS), lambda i: (0, 0)),
        ],
        out_specs=[
            pl.BlockSpec((1, NODES_PER_GRAPH, D), lambda i: (i, 0, 0)),
            pl.BlockSpec((1, 1, NCLS), lambda i: (i, 0, 0)),
        ],
        out_shape=[
            jax.ShapeDtypeStruct((NUM_GRAPHS, NODES_PER_GRAPH, D), jnp.float32),
            jax.ShapeDtypeStruct((NUM_GRAPHS, 1, NCLS), jnp.float32),
        ],
    )(h, W_cls, b_cls2)


def _sc_edge_body(trans_hbm, packed_hbm, out_hbm,
                  packed_v, idxs_v, dsts_v, rows_v, acc_sh, gsem, ssem):
    c = lax.axis_index("c")
    s = lax.axis_index("s")

    def _z(i, _):
        for j in range(DH // 16):
            rows_v[0, i, pl.ds(j * 16, 16)] = jnp.zeros((16,), jnp.float32)
        return 0
    lax.fori_loop(0, CHUNK, _z, 0)
    base = pl.multiple_of(s * SLAB, 8)

    def _zk(k, _):
        pltpu.sync_copy(rows_v.at[0],
                        acc_sh.at[pl.ds(base + k * CHUNK, CHUNK)])
        return 0
    lax.fori_loop(0, SLAB // CHUNK, _zk, 0)

    @pl.when(s < 15)
    def _zremf():
        zr = SLAB % CHUNK
        pltpu.sync_copy(rows_v.at[0, pl.ds(0, zr)],
                        acc_sh.at[pl.ds(base + SLAB - zr, zr)])

    @pl.when(s == 15)
    def _zremt():
        zr = ZTAIL % CHUNK
        pltpu.sync_copy(rows_v.at[0, pl.ds(0, zr)],
                        acc_sh.at[pl.ds(15 * SLAB + ZTAIL - zr, zr)])

    pltpu.sync_copy(packed_hbm.at[s], packed_v)

    plsc.subcore_barrier()

    def _unpack(jj, slot):
        for i in range(CHUNK // 16):
            v = packed_v[jj, pl.ds(i * 16, 16)]
            key16 = jnp.bitwise_and(v, 0xFFFF)
            idxs_v[slot, pl.ds(i * 16, 16)] = key16 * 2 + c
            dsts_v[slot, pl.ds(i * 16, 16)] = lax.shift_right_logical(v, 16)

    def _fire_gather(jj, slot):
        pltpu.async_copy(trans_hbm.at[idxs_v.at[slot]], rows_v.at[slot], gsem)

    def _prime(p, _):
        _unpack(p, p)
        _fire_gather(p, p)
        return 0
    lax.fori_loop(0, PREF, _prime, 0)

    def _chunk(j, _):
        r = j % NBUF
        rn = (j + PREF) % NBUF
        pltpu.make_async_copy(trans_hbm.at[idxs_v.at[r]], rows_v.at[r],
                              gsem).wait()
        pltpu.async_copy(rows_v.at[r], acc_sh.at[dsts_v.at[r]], ssem,
                         add=True)

        @pl.when(j + PREF < NCH)
        def _prefetch():
            @pl.when(j + PREF - NBUF >= 0)
            def _reclaim():
                pltpu.make_async_copy(rows_v.at[rn],
                                      acc_sh.at[dsts_v.at[rn]], ssem).wait()
            _unpack(j + PREF, rn)
            _fire_gather(j + PREF, rn)
        return 0
    lax.fori_loop(0, NCH, _chunk, 0)

    def _drain(t, _):
        slot = (NCH - NBUF + t) % NBUF
        pltpu.make_async_copy(rows_v.at[slot], acc_sh.at[dsts_v.at[slot]],
                              ssem).wait()
        return 0
    lax.fori_loop(0, NBUF, _drain, 0)

    plsc.subcore_barrier()

    @pl.when(s < 15)
    def _full():
        pltpu.sync_copy(acc_sh.at[pl.ds(base, SLAB)],
                        out_hbm.at[c, pl.ds(base, SLAB)])

    @pl.when(s == 15)
    def _tail():
        pltpu.sync_copy(acc_sh.at[pl.ds(15 * SLAB, TAIL)],
                        out_hbm.at[c, pl.ds(15 * SLAB, TAIL)])


_sc_edge = functools.partial(
    pl.kernel,
    out_type=jax.ShapeDtypeStruct((2, N, DH), jnp.float32),
    mesh=plsc.VectorSubcoreMesh(core_axis_name="c", subcore_axis_name="s"),
    compiler_params=pltpu.CompilerParams(use_tc_tiling_on_sc=False),
    scratch_types=[
        pltpu.VMEM((NCH, CHUNK), jnp.int32),
        pltpu.VMEM((NBUF, CHUNK), jnp.int32),
        pltpu.VMEM((NBUF, CHUNK), jnp.int32),
        pltpu.VMEM((NBUF, CHUNK, DH), jnp.float32),
        pltpu.VMEM_SHARED((ACC_ROWS, DH), jnp.float32),
        pltpu.SemaphoreType.DMA,
        pltpu.SemaphoreType.DMA,
    ],
)(_sc_edge_body)


def kernel(x, edge_index, edge_types, W_edge, b_edge, W_ih, W_hh, b_ih, b_hh,
           W_cls, b_cls):
    src = edge_index[0]
    dst = edge_index[1]
    key = edge_types * N + src
    packed = key + (dst << 16)
    packed_r = jnp.concatenate(
        [packed, jnp.full((EPAD - E,), N << 16, jnp.int32)]
    ).reshape(16, NCH, CHUNK)

    W_ih_T = W_ih.T
    W_hh_T = W_hh.T
    b_ih2 = b_ih[None, :]
    b_hh2 = b_hh[None, :]
    b_cls2 = b_cls[None, :]

    h = x
    trans = _first_trans(x, W_edge, b_edge)
    for step in range(STEPS):
        part = _sc_edge(trans.reshape(2 * ETYPES * N, DH), packed_r)
        h, trans = _gru_step(part, h, W_ih_T, W_hh_T, b_ih2,
                             b_hh2, W_edge, b_edge,
                             with_trans=(step < STEPS - 1))
    h_i, ggnn_sum = _final(h, W_cls, b_cls2)
    return (ggnn_sum.reshape(NUM_GRAPHS, NCLS), h_i)


# speedup vs baseline: 1.7813x; 1.0002x over previous
import functools

import jax
import jax.numpy as jnp
from jax import lax
from jax.experimental import pallas as pl
from jax.experimental.pallas import tpu as pltpu
from jax.experimental.pallas import tpu_sc as plsc

N = 10000
E = 320000
D = 128
ETYPES = 4
STEPS = 8
NUM_GRAPHS = 10
NODES_PER_GRAPH = 1000
NCLS = 46

BN = 2000

DH = D // 2
CHUNK = 128
NCH = 157
NBUF = 7
PREF = 6
EPAD = 16 * NCH * CHUNK
SLAB = 632
ACC_ROWS = N + 8
ZTAIL = ACC_ROWS - 15 * SLAB
TAIL = N - 15 * SLAB


def _first_trans_body(x_ref, we_ref, be_ref, trans_ref):
    h = x_ref[...]
    for t in range(ETYPES):
        tr = jnp.dot(h, we_ref[t], preferred_element_type=jnp.float32)
        trans_ref[t] = tr + be_ref[t][None, :]


def _first_trans(x, W_edge, b_edge):
    grid = (N // BN,)
    return pl.pallas_call(
        _first_trans_body,
        grid=grid,
        in_specs=[
            pl.BlockSpec((BN, D), lambda i: (i, 0)),
            pl.BlockSpec((ETYPES, D, D), lambda i: (0, 0, 0)),
            pl.BlockSpec((ETYPES, D), lambda i: (0, 0)),
        ],
        out_specs=pl.BlockSpec((ETYPES, BN, D), lambda i: (0, i, 0)),
        out_shape=jax.ShapeDtypeStruct((ETYPES, N, D), jnp.float32),
    )(x, W_edge, b_edge)


def _gru_body(with_trans, part_ref, h_ref, wih_ref, whh_ref, bih_ref,
              bhh_ref, we_ref, be_ref, h_new_ref, trans_ref=None):
    a = jnp.concatenate([part_ref[0], part_ref[1]], axis=1)
    h = h_ref[...]
    gi = jnp.dot(a, wih_ref[...], preferred_element_type=jnp.float32) + bih_ref[...]
    gh = jnp.dot(h, whh_ref[...], preferred_element_type=jnp.float32) + bhh_ref[...]
    r = jax.nn.sigmoid(gi[:, :D] + gh[:, :D])
    z = jax.nn.sigmoid(gi[:, D:2 * D] + gh[:, D:2 * D])
    n = jnp.tanh(gi[:, 2 * D:] + r * gh[:, 2 * D:])
    h_new = (1.0 - z) * n + z * h
    h_new_ref[...] = h_new
    if with_trans:
        for t in range(ETYPES):
            tr = jnp.dot(h_new, we_ref[t], preferred_element_type=jnp.float32)
            trans_ref[t] = tr + be_ref[t][None, :]


def _gru_step(part, h, W_ih_T, W_hh_T, b_ih2, b_hh2, W_edge, b_edge,
              with_trans):
    grid = (N // BN,)
    out_shape = [jax.ShapeDtypeStruct((N, D), jnp.float32)]
    out_specs = [pl.BlockSpec((BN, D), lambda i: (i, 0))]
    if with_trans:
        out_shape.append(jax.ShapeDtypeStruct((ETYPES, N, D), jnp.float32))
        out_specs.append(pl.BlockSpec((ETYPES, BN, D), lambda i: (0, i, 0)))
    res = pl.pallas_call(
        functools.partial(_gru_body, with_trans),
        grid=grid,
        in_specs=[
            pl.BlockSpec((2, BN, DH), lambda i: (0, i, 0)),
            pl.BlockSpec((BN, D), lambda i: (i, 0)),
            pl.BlockSpec((D, 3 * D), lambda i: (0, 0)),
            pl.BlockSpec((D, 3 * D), lambda i: (0, 0)),
            pl.BlockSpec((1, 3 * D), lambda i: (0, 0)),
            pl.BlockSpec((1, 3 * D), lambda i: (0, 0)),
            pl.BlockSpec((ETYPES, D, D), lambda i: (0, 0, 0)),
            pl.BlockSpec((ETYPES, D), lambda i: (0, 0)),
        ],
        out_specs=out_specs,
        out_shape=out_shape,
    )(part, h, W_ih_T, W_hh_T, b_ih2, b_hh2, W_edge, b_edge)
    if with_trans:
        return res[0], res[1]
    return res[0], None


def _final_body(h_ref, wcls_ref, bcls_ref, hi_ref, cls_ref):
    y = jnp.tanh(h_ref[...])
    hi_ref[0] = y
    s = jnp.sum(y, axis=0, keepdims=True)
    cls_ref[0] = jnp.dot(s, wcls_ref[...].T,
                         preferred_element_type=jnp.float32) + bcls_ref[...]


def _final(h, W_cls, b_cls2):
    grid = (NUM_GRAPHS,)
    return pl.pallas_call(
        _final_body,
        grid=grid,
        in_specs=[
            pl.BlockSpec((NODES_PER_GRAPH, D), lambda i: (i, 0)),
            pl.BlockSpec((NCLS, D), lambda i: (0, 0)),
            pl.BlockSpec((1, NCLS), lambda i: (0, 0)),
        ],
        out_specs=[
            pl.BlockSpec((1, NODES_PER_GRAPH, D), lambda i: (i, 0, 0)),
            pl.BlockSpec((1, 1, NCLS), lambda i: (i, 0, 0)),
        ],
        out_shape=[
            jax.ShapeDtypeStruct((NUM_GRAPHS, NODES_PER_GRAPH, D), jnp.float32),
            jax.ShapeDtypeStruct((NUM_GRAPHS, 1, NCLS), jnp.float32),
        ],
    )(h, W_cls, b_cls2)


def _sc_edge_body(trans_hbm, packed_hbm, out_hbm,
                  packed_v, idxs_v, dsts_v, rows_v, acc_sh, gsem, ssem):
    c = lax.axis_index("c")
    s = lax.axis_index("s")

    def _z(i, _):
        for j in range(DH // 16):
            rows_v[0, i, pl.ds(j * 16, 16)] = jnp.zeros((16,), jnp.float32)
        return 0
    lax.fori_loop(0, CHUNK, _z, 0)
    base = pl.multiple_of(s * SLAB, 8)

    def _zk(k, _):
        pltpu.sync_copy(rows_v.at[0],
                        acc_sh.at[pl.ds(base + k * CHUNK, CHUNK)])
        return 0
    lax.fori_loop(0, SLAB // CHUNK, _zk, 0)

    @pl.when(s < 15)
    def _zremf():
        zr = SLAB % CHUNK
        pltpu.sync_copy(rows_v.at[0, pl.ds(0, zr)],
                        acc_sh.at[pl.ds(base + SLAB - zr, zr)])

    @pl.when(s == 15)
    def _zremt():
        zr = ZTAIL % CHUNK
        pltpu.sync_copy(rows_v.at[0, pl.ds(0, zr)],
                        acc_sh.at[pl.ds(15 * SLAB + ZTAIL - zr, zr)])

    pltpu.sync_copy(packed_hbm.at[s], packed_v)

    plsc.subcore_barrier()

    def _unpack(jj, slot):
        for i in range(CHUNK // 16):
            v = packed_v[jj, pl.ds(i * 16, 16)]
            key16 = jnp.bitwise_and(v, 0xFFFF)
            idxs_v[slot, pl.ds(i * 16, 16)] = key16 * 2 + c
            dsts_v[slot, pl.ds(i * 16, 16)] = lax.shift_right_logical(v, 16)

    def _fire_gather(jj, slot):
        pltpu.async_copy(trans_hbm.at[idxs_v.at[slot]], rows_v.at[slot], gsem)

    def _prime(p, _):
        _unpack(p, p)
        _fire_gather(p, p)
        return 0
    lax.fori_loop(0, PREF, _prime, 0)

    def _chunk(j, _):
        r = j % NBUF
        rn = (j + PREF) % NBUF
        pltpu.make_async_copy(trans_hbm.at[idxs_v.at[r]], rows_v.at[r],
                              gsem).wait()
        pltpu.async_copy(rows_v.at[r], acc_sh.at[dsts_v.at[r]], ssem,
                         add=True)

        @pl.when(j + PREF < NCH)
        def _prefetch():
            @pl.when(j + PREF - NBUF >= 0)
            def _reclaim():
                pltpu.make_async_copy(rows_v.at[rn],
                                      acc_sh.at[dsts_v.at[rn]], ssem).wait()
            _unpack(j + PREF, rn)
            _fire_gather(j + PREF, rn)
        return 0
    lax.fori_loop(0, NCH, _chunk, 0)

    def _drain(t, _):
        slot = (NCH - NBUF + t) % NBUF
        pltpu.make_async_copy(rows_v.at[slot], acc_sh.at[dsts_v.at[slot]],
                              ssem).wait()
        return 0
    lax.fori_loop(0, NBUF, _drain, 0)

    plsc.subcore_barrier()

    @pl.when(s < 15)
    def _full():
        pltpu.sync_copy(acc_sh.at[pl.ds(base, SLAB)],
                        out_hbm.at[c, pl.ds(base, SLAB)])

    @pl.when(s == 15)
    def _tail():
        pltpu.sync_copy(acc_sh.at[pl.ds(15 * SLAB, TAIL)],
                        out_hbm.at[c, pl.ds(15 * SLAB, TAIL)])


@functools.cache
def _sc_edge_kernel():
    return functools.partial(
        pl.kernel,
        out_type=jax.ShapeDtypeStruct((2, N, DH), jnp.float32),
        mesh=plsc.VectorSubcoreMesh(core_axis_name="c",
                                    subcore_axis_name="s"),
        compiler_params=pltpu.CompilerParams(use_tc_tiling_on_sc=False),
        scratch_types=[
            pltpu.VMEM((NCH, CHUNK), jnp.int32),
            pltpu.VMEM((NBUF, CHUNK), jnp.int32),
            pltpu.VMEM((NBUF, CHUNK), jnp.int32),
            pltpu.VMEM((NBUF, CHUNK, DH), jnp.float32),
            pltpu.VMEM_SHARED((ACC_ROWS, DH), jnp.float32),
            pltpu.SemaphoreType.DMA,
            pltpu.SemaphoreType.DMA,
        ],
    )(_sc_edge_body)


def _sc_edge(table, packed_r):
    return _sc_edge_kernel()(table, packed_r)


def kernel(x, edge_index, edge_types, W_edge, b_edge, W_ih, W_hh, b_ih, b_hh,
           W_cls, b_cls):
    src = edge_index[0]
    dst = edge_index[1]
    key = edge_types * N + src
    packed = key + (dst << 16)
    packed_r = jnp.concatenate(
        [packed, jnp.full((EPAD - E,), N << 16, jnp.int32)]
    ).reshape(16, NCH, CHUNK)

    W_ih_T = W_ih.T
    W_hh_T = W_hh.T
    b_ih2 = b_ih[None, :]
    b_hh2 = b_hh[None, :]
    b_cls2 = b_cls[None, :]

    h = x
    trans = _first_trans(x, W_edge, b_edge)
    for step in range(STEPS):
        part = _sc_edge(trans.reshape(2 * ETYPES * N, DH), packed_r)
        h, trans = _gru_step(part, h, W_ih_T, W_hh_T, b_ih2,
                             b_hh2, W_edge, b_edge,
                             with_trans=(step < STEPS - 1))
    h_i, ggnn_sum = _final(h, W_cls, b_cls2)
    return (ggnn_sum.reshape(NUM_GRAPHS, NCLS), h_i)
